# Initial kernel scaffold; baseline (speedup 1.0000x reference)
import jax
import jax.numpy as jnp
from jax.experimental import pallas as pl


def kernel(xyz1, xyz2):
    raise NotImplementedError("write your pallas kernel here")


# speedup vs baseline: 1.9193x; 1.9193x over previous
import functools

import jax
import jax.numpy as jnp
from jax import lax
from jax.experimental import pallas as pl
from jax.experimental.pallas import tpu as pltpu


def _chamfer_body(x1_ref, x2n_ref, sq1_ref, sq2_ref, idx1_ref, idx2_ref,
                  cmin_ref, cidx_ref, *, tn: int, m: int, nblk: int):
    k = pl.program_id(1)

    x1t = x1_ref[0]
    x2n = x2n_ref[0]
    sq1t = sq1_ref[0]
    sq2r = sq2_ref[0]

    inner2 = lax.dot_general(x1t, x2n, (((1,), (1,)), ((), ())),
                             preferred_element_type=jnp.float32)
    d = (sq1t + sq2r) + inner2

    big = jnp.int32(m)

    rmin = jnp.min(d, axis=1, keepdims=True)
    lane_iota = lax.broadcasted_iota(jnp.int32, (tn, m), 1)
    ridx = jnp.min(jnp.where(d == rmin, lane_iota, big),
                   axis=1, keepdims=True)
    idx1_ref[0] = ridx

    cmin_t = jnp.min(d, axis=0, keepdims=True)
    sub_iota = lax.broadcasted_iota(jnp.int32, (tn, m), 0)
    cidx_t = jnp.min(jnp.where(d == cmin_t, sub_iota, big),
                     axis=0, keepdims=True) + k * tn

    @pl.when(k == 0)
    def _():
        cmin_ref[...] = cmin_t
        cidx_ref[...] = cidx_t

    @pl.when(k > 0)
    def _():
        prev_min = cmin_ref[...]
        prev_idx = cidx_ref[...]
        upd = cmin_t < prev_min
        cmin_ref[...] = jnp.where(upd, cmin_t, prev_min)
        cidx_ref[...] = jnp.where(upd, cidx_t, prev_idx)

    @pl.when(k == nblk - 1)
    def _():
        idx2_ref[0] = cidx_ref[...]


@jax.jit
def kernel(xyz1, xyz2):
    b, n, _ = xyz1.shape
    m = xyz2.shape[1]
    tn = 256
    nblk = n // tn

    sq1 = jnp.sum(xyz1 * xyz1, axis=-1)[..., None]
    sq2 = jnp.sum(xyz2 * xyz2, axis=-1)[:, None, :]
    x2n = xyz2 * jnp.float32(-2.0)

    grid = (b, nblk)
    idx1, idx2 = pl.pallas_call(
        functools.partial(_chamfer_body, tn=tn, m=m, nblk=nblk),
        grid=grid,
        in_specs=[
            pl.BlockSpec((1, tn, 3), lambda bi, ki: (bi, ki, 0)),
            pl.BlockSpec((1, m, 3), lambda bi, ki: (bi, 0, 0)),
            pl.BlockSpec((1, tn, 1), lambda bi, ki: (bi, ki, 0)),
            pl.BlockSpec((1, 1, m), lambda bi, ki: (bi, 0, 0)),
        ],
        out_specs=[
            pl.BlockSpec((1, tn, 1), lambda bi, ki: (bi, ki, 0)),
            pl.BlockSpec((1, 1, m), lambda bi, ki: (bi, 0, 0)),
        ],
        out_shape=[
            jax.ShapeDtypeStruct((b, n, 1), jnp.int32),
            jax.ShapeDtypeStruct((b, 1, m), jnp.int32),
        ],
        scratch_shapes=[
            pltpu.VMEM((1, m), jnp.float32),
            pltpu.VMEM((1, m), jnp.int32),
        ],
        compiler_params=pltpu.CompilerParams(
            dimension_semantics=("arbitrary", "arbitrary"),
        ),
    )(xyz1, x2n, sq1, sq2)

    return idx1[..., 0], idx2[:, 0, :]


# speedup vs baseline: 2.3345x; 1.2164x over previous
import functools

import jax
import jax.numpy as jnp
from jax import lax
from jax.experimental import pallas as pl
from jax.experimental.pallas import tpu as pltpu


def _chamfer_body(x1_ref, x2n_ref, sq1_ref, sq2_ref, idx1_ref, idx2_ref,
                  cmin_ref, cidx_ref, *, tn: int, m: int, nblk: int):
    k = pl.program_id(1)

    x1t = x1_ref[0]
    x2n = x2n_ref[0]
    sq1t = sq1_ref[0]
    sq2r = sq2_ref[0]

    inner2 = lax.dot_general(x1t, x2n, (((1,), (1,)), ((), ())),
                             preferred_element_type=jnp.float32)
    d = (sq1t + sq2r) + inner2

    big = jnp.int32(m)

    lb = 128
    nlb = m // lb
    run_v = d[:, 0:lb]
    run_l = jnp.zeros((tn, lb), jnp.int32)
    for l in range(1, nlb):
        x = d[:, l * lb:(l + 1) * lb]
        win = x < run_v
        run_v = jnp.where(win, x, run_v)
        run_l = jnp.where(win, l, run_l)
    rmin = jnp.min(run_v, axis=1, keepdims=True)
    lane_iota = lax.broadcasted_iota(jnp.int32, (tn, lb), 1)
    ridx = jnp.min(jnp.where(run_v == rmin, run_l * lb + lane_iota, big),
                   axis=1, keepdims=True)
    idx1_ref[0] = ridx

    nsc = tn // 8
    d3 = d.reshape(nsc, 8, m)
    run_v2 = d3[0]
    run_c = jnp.zeros((8, m), jnp.int32)
    for c in range(1, nsc):
        x = d3[c]
        win = x < run_v2
        run_v2 = jnp.where(win, x, run_v2)
        run_c = jnp.where(win, c, run_c)
    cmin_t = jnp.min(run_v2, axis=0, keepdims=True)
    sub_iota = lax.broadcasted_iota(jnp.int32, (8, m), 0)
    cidx_t = jnp.min(jnp.where(run_v2 == cmin_t, run_c * 8 + sub_iota, big),
                     axis=0, keepdims=True) + k * tn

    @pl.when(k == 0)
    def _():
        cmin_ref[...] = cmin_t
        cidx_ref[...] = cidx_t

    @pl.when(k > 0)
    def _():
        prev_min = cmin_ref[...]
        prev_idx = cidx_ref[...]
        upd = cmin_t < prev_min
        cmin_ref[...] = jnp.where(upd, cmin_t, prev_min)
        cidx_ref[...] = jnp.where(upd, cidx_t, prev_idx)

    @pl.when(k == nblk - 1)
    def _():
        idx2_ref[0] = cidx_ref[...]


@jax.jit
def kernel(xyz1, xyz2):
    b, n, _ = xyz1.shape
    m = xyz2.shape[1]
    tn = 256
    nblk = n // tn

    sq1 = jnp.sum(xyz1 * xyz1, axis=-1)[..., None]
    sq2 = jnp.sum(xyz2 * xyz2, axis=-1)[:, None, :]
    x2n = xyz2 * jnp.float32(-2.0)

    grid = (b, nblk)
    idx1, idx2 = pl.pallas_call(
        functools.partial(_chamfer_body, tn=tn, m=m, nblk=nblk),
        grid=grid,
        in_specs=[
            pl.BlockSpec((1, tn, 3), lambda bi, ki: (bi, ki, 0)),
            pl.BlockSpec((1, m, 3), lambda bi, ki: (bi, 0, 0)),
            pl.BlockSpec((1, tn, 1), lambda bi, ki: (bi, ki, 0)),
            pl.BlockSpec((1, 1, m), lambda bi, ki: (bi, 0, 0)),
        ],
        out_specs=[
            pl.BlockSpec((1, tn, 1), lambda bi, ki: (bi, ki, 0)),
            pl.BlockSpec((1, 1, m), lambda bi, ki: (bi, 0, 0)),
        ],
        out_shape=[
            jax.ShapeDtypeStruct((b, n, 1), jnp.int32),
            jax.ShapeDtypeStruct((b, 1, m), jnp.int32),
        ],
        scratch_shapes=[
            pltpu.VMEM((1, m), jnp.float32),
            pltpu.VMEM((1, m), jnp.int32),
        ],
        compiler_params=pltpu.CompilerParams(
            dimension_semantics=("arbitrary", "arbitrary"),
        ),
    )(xyz1, x2n, sq1, sq2)

    return idx1[..., 0], idx2[:, 0, :]


# speedup vs baseline: 2.7185x; 1.1645x over previous
import functools

import jax
import jax.numpy as jnp
from jax import lax
from jax.experimental import pallas as pl
from jax.experimental.pallas import tpu as pltpu


def _chamfer_body(x1_ref, x2n_ref, sq1_ref, sq2_ref, idx1_ref, idx2_ref,
                  cmin_ref, cidx_ref, *, tn: int, m: int, nblk: int):
    k = pl.program_id(1)

    x1t = x1_ref[0]
    x2n = x2n_ref[0]
    sq1t = sq1_ref[0]
    sq2r = sq2_ref[0]

    inner2 = lax.dot_general(x1t, x2n, (((1,), (1,)), ((), ())),
                             preferred_element_type=jnp.float32)
    d = (sq1t + sq2r) + inner2

    big = jnp.int32(m)

    lb = 128
    nlb = m // lb
    run_v = d[:, 0:lb]
    run_l = jnp.zeros((tn, lb), jnp.int32)
    for l in range(1, nlb):
        x = d[:, l * lb:(l + 1) * lb]
        win = x < run_v
        run_v = jnp.where(win, x, run_v)
        run_l = jnp.where(win, l, run_l)
    rmin = jnp.min(run_v, axis=1, keepdims=True)
    lane_iota = lax.broadcasted_iota(jnp.int32, (tn, lb), 1)
    ridx = jnp.min(jnp.where(run_v == rmin, run_l * lb + lane_iota, big),
                   axis=1, keepdims=True)
    idx1_ref[0] = ridx

    nsc = tn // 8
    d3 = d.reshape(nsc, 8, m)
    run_v2 = d3[0]
    run_c = jnp.zeros((8, m), jnp.int32)
    for c in range(1, nsc):
        x = d3[c]
        win = x < run_v2
        run_v2 = jnp.where(win, x, run_v2)
        run_c = jnp.where(win, c, run_c)
    cmin_t = jnp.min(run_v2, axis=0, keepdims=True)
    sub_iota = lax.broadcasted_iota(jnp.int32, (8, m), 0)
    cidx_t = jnp.min(jnp.where(run_v2 == cmin_t, run_c * 8 + sub_iota, big),
                     axis=0, keepdims=True) + k * tn

    @pl.when(k == 0)
    def _():
        cmin_ref[...] = cmin_t
        cidx_ref[...] = cidx_t

    @pl.when(k > 0)
    def _():
        prev_min = cmin_ref[...]
        prev_idx = cidx_ref[...]
        upd = cmin_t < prev_min
        cmin_ref[...] = jnp.where(upd, cmin_t, prev_min)
        cidx_ref[...] = jnp.where(upd, cidx_t, prev_idx)

    @pl.when(k == nblk - 1)
    def _():
        idx2_ref[0] = cidx_ref[...]


@jax.jit
def kernel(xyz1, xyz2):
    b, n, _ = xyz1.shape
    m = xyz2.shape[1]
    tn = 512
    nblk = n // tn

    sq1 = jnp.sum(xyz1 * xyz1, axis=-1)[..., None]
    sq2 = jnp.sum(xyz2 * xyz2, axis=-1)[:, None, :]
    x2n = xyz2 * jnp.float32(-2.0)

    grid = (b, nblk)
    idx1, idx2 = pl.pallas_call(
        functools.partial(_chamfer_body, tn=tn, m=m, nblk=nblk),
        grid=grid,
        in_specs=[
            pl.BlockSpec((1, tn, 3), lambda bi, ki: (bi, ki, 0)),
            pl.BlockSpec((1, m, 3), lambda bi, ki: (bi, 0, 0)),
            pl.BlockSpec((1, tn, 1), lambda bi, ki: (bi, ki, 0)),
            pl.BlockSpec((1, 1, m), lambda bi, ki: (bi, 0, 0)),
        ],
        out_specs=[
            pl.BlockSpec((1, tn, 1), lambda bi, ki: (bi, ki, 0)),
            pl.BlockSpec((1, 1, m), lambda bi, ki: (bi, 0, 0)),
        ],
        out_shape=[
            jax.ShapeDtypeStruct((b, n, 1), jnp.int32),
            jax.ShapeDtypeStruct((b, 1, m), jnp.int32),
        ],
        scratch_shapes=[
            pltpu.VMEM((1, m), jnp.float32),
            pltpu.VMEM((1, m), jnp.int32),
        ],
        compiler_params=pltpu.CompilerParams(
            dimension_semantics=("arbitrary", "arbitrary"),
        ),
    )(xyz1, x2n, sq1, sq2)

    return idx1[..., 0], idx2[:, 0, :]


# speedup vs baseline: 2.9029x; 1.0679x over previous
import functools

import jax
import jax.numpy as jnp
from jax import lax
from jax.experimental import pallas as pl
from jax.experimental.pallas import tpu as pltpu


def _chamfer_body(x1_ref, x2n_ref, sq1_ref, sq2_ref, idx1_ref, idx2_ref,
                  cmin_ref, cidx_ref, *, tn: int, m: int, nblk: int):
    k = pl.program_id(1)

    x1t = x1_ref[0]
    x2n = x2n_ref[0]
    sq1t = sq1_ref[0]
    sq2r = sq2_ref[0]

    inner2 = lax.dot_general(x1t, x2n, (((1,), (1,)), ((), ())),
                             preferred_element_type=jnp.float32)
    d = (sq1t + sq2r) + inner2

    big = jnp.int32(m)

    lb = 128
    nlb = m // lb
    run_v = d[:, 0:lb]
    run_l = jnp.zeros((tn, lb), jnp.int32)
    for l in range(1, nlb):
        x = d[:, l * lb:(l + 1) * lb]
        win = x < run_v
        run_v = jnp.where(win, x, run_v)
        run_l = jnp.where(win, l, run_l)
    rmin = jnp.min(run_v, axis=1, keepdims=True)
    lane_iota = lax.broadcasted_iota(jnp.int32, (tn, lb), 1)
    ridx = jnp.min(jnp.where(run_v == rmin, run_l * lb + lane_iota, big),
                   axis=1, keepdims=True)
    idx1_ref[0] = ridx

    nsc = tn // 8
    d3 = d.reshape(nsc, 8, m)
    run_v2 = d3[0]
    run_c = jnp.zeros((8, m), jnp.int32)
    for c in range(1, nsc):
        x = d3[c]
        win = x < run_v2
        run_v2 = jnp.where(win, x, run_v2)
        run_c = jnp.where(win, c, run_c)
    cmin_t = jnp.min(run_v2, axis=0, keepdims=True)
    sub_iota = lax.broadcasted_iota(jnp.int32, (8, m), 0)
    cidx_t = jnp.min(jnp.where(run_v2 == cmin_t, run_c * 8 + sub_iota, big),
                     axis=0, keepdims=True) + k * tn

    @pl.when(k == 0)
    def _():
        cmin_ref[...] = cmin_t
        cidx_ref[...] = cidx_t

    @pl.when(k > 0)
    def _():
        prev_min = cmin_ref[...]
        prev_idx = cidx_ref[...]
        upd = cmin_t < prev_min
        cmin_ref[...] = jnp.where(upd, cmin_t, prev_min)
        cidx_ref[...] = jnp.where(upd, cidx_t, prev_idx)

    @pl.when(k == nblk - 1)
    def _():
        idx2_ref[0] = cidx_ref[...]


@jax.jit
def kernel(xyz1, xyz2):
    b, n, _ = xyz1.shape
    m = xyz2.shape[1]
    tn = 1024
    nblk = n // tn

    sq1 = jnp.sum(xyz1 * xyz1, axis=-1)[..., None]
    sq2 = jnp.sum(xyz2 * xyz2, axis=-1)[:, None, :]
    x2n = xyz2 * jnp.float32(-2.0)

    grid = (b, nblk)
    idx1, idx2 = pl.pallas_call(
        functools.partial(_chamfer_body, tn=tn, m=m, nblk=nblk),
        grid=grid,
        in_specs=[
            pl.BlockSpec((1, tn, 3), lambda bi, ki: (bi, ki, 0)),
            pl.BlockSpec((1, m, 3), lambda bi, ki: (bi, 0, 0)),
            pl.BlockSpec((1, tn, 1), lambda bi, ki: (bi, ki, 0)),
            pl.BlockSpec((1, 1, m), lambda bi, ki: (bi, 0, 0)),
        ],
        out_specs=[
            pl.BlockSpec((1, tn, 1), lambda bi, ki: (bi, ki, 0)),
            pl.BlockSpec((1, 1, m), lambda bi, ki: (bi, 0, 0)),
        ],
        out_shape=[
            jax.ShapeDtypeStruct((b, n, 1), jnp.int32),
            jax.ShapeDtypeStruct((b, 1, m), jnp.int32),
        ],
        scratch_shapes=[
            pltpu.VMEM((1, m), jnp.float32),
            pltpu.VMEM((1, m), jnp.int32),
        ],
        compiler_params=pltpu.CompilerParams(
            dimension_semantics=("arbitrary", "arbitrary"),
        ),
    )(xyz1, x2n, sq1, sq2)

    return idx1[..., 0], idx2[:, 0, :]


# speedup vs baseline: 3.1085x; 1.0708x over previous
import functools

import jax
import jax.numpy as jnp
from jax import lax
from jax.experimental import pallas as pl
from jax.experimental.pallas import tpu as pltpu


def _chamfer_body(x1_ref, x2n_ref, sq1_ref, sq2_ref, idx1_ref, idx2_ref,
                  cmin_ref, cidx_ref, *, tn: int, m: int, nblk: int):
    k = pl.program_id(1)

    x1t = x1_ref[0]
    x2n = x2n_ref[0]
    sq1t = sq1_ref[0]
    sq2r = sq2_ref[0]

    inner2 = lax.dot_general(x1t, x2n, (((1,), (1,)), ((), ())),
                             preferred_element_type=jnp.float32)
    d = (sq1t + sq2r) + inner2

    big = jnp.int32(m)

    lb = 128
    nlb = m // lb
    run_v = d[:, 0:lb]
    run_l = jnp.zeros((tn, lb), jnp.int32)
    for l in range(1, nlb):
        x = d[:, l * lb:(l + 1) * lb]
        win = x < run_v
        run_v = jnp.where(win, x, run_v)
        run_l = jnp.where(win, l, run_l)
    rmin = jnp.min(run_v, axis=1, keepdims=True)
    lane_iota = lax.broadcasted_iota(jnp.int32, (tn, lb), 1)
    ridx = jnp.min(jnp.where(run_v == rmin, run_l * lb + lane_iota, big),
                   axis=1, keepdims=True)
    idx1_ref[0] = ridx

    nsc = tn // 8
    d3 = d.reshape(nsc, 8, m)
    run_v2 = d3[0]
    run_c = jnp.zeros((8, m), jnp.int32)
    for c in range(1, nsc):
        x = d3[c]
        win = x < run_v2
        run_v2 = jnp.where(win, x, run_v2)
        run_c = jnp.where(win, c, run_c)
    cmin_t = jnp.min(run_v2, axis=0, keepdims=True)
    sub_iota = lax.broadcasted_iota(jnp.int32, (8, m), 0)
    cidx_t = jnp.min(jnp.where(run_v2 == cmin_t, run_c * 8 + sub_iota, big),
                     axis=0, keepdims=True) + k * tn

    @pl.when(k == 0)
    def _():
        cmin_ref[...] = cmin_t
        cidx_ref[...] = cidx_t

    @pl.when(k > 0)
    def _():
        prev_min = cmin_ref[...]
        prev_idx = cidx_ref[...]
        upd = cmin_t < prev_min
        cmin_ref[...] = jnp.where(upd, cmin_t, prev_min)
        cidx_ref[...] = jnp.where(upd, cidx_t, prev_idx)

    @pl.when(k == nblk - 1)
    def _():
        idx2_ref[0] = cidx_ref[...]


@jax.jit
def kernel(xyz1, xyz2):
    b, n, _ = xyz1.shape
    m = xyz2.shape[1]
    tn = 2048
    nblk = n // tn

    sq1 = jnp.sum(xyz1 * xyz1, axis=-1)[..., None]
    sq2 = jnp.sum(xyz2 * xyz2, axis=-1)[:, None, :]
    x2n = xyz2 * jnp.float32(-2.0)

    grid = (b, nblk)
    idx1, idx2 = pl.pallas_call(
        functools.partial(_chamfer_body, tn=tn, m=m, nblk=nblk),
        grid=grid,
        in_specs=[
            pl.BlockSpec((1, tn, 3), lambda bi, ki: (bi, ki, 0)),
            pl.BlockSpec((1, m, 3), lambda bi, ki: (bi, 0, 0)),
            pl.BlockSpec((1, tn, 1), lambda bi, ki: (bi, ki, 0)),
            pl.BlockSpec((1, 1, m), lambda bi, ki: (bi, 0, 0)),
        ],
        out_specs=[
            pl.BlockSpec((1, tn, 1), lambda bi, ki: (bi, ki, 0)),
            pl.BlockSpec((1, 1, m), lambda bi, ki: (bi, 0, 0)),
        ],
        out_shape=[
            jax.ShapeDtypeStruct((b, n, 1), jnp.int32),
            jax.ShapeDtypeStruct((b, 1, m), jnp.int32),
        ],
        scratch_shapes=[
            pltpu.VMEM((1, m), jnp.float32),
            pltpu.VMEM((1, m), jnp.int32),
        ],
        compiler_params=pltpu.CompilerParams(
            dimension_semantics=("arbitrary", "arbitrary"),
        ),
    )(xyz1, x2n, sq1, sq2)

    return idx1[..., 0], idx2[:, 0, :]


# speedup vs baseline: 3.4434x; 1.1077x over previous
import functools

import jax
import jax.numpy as jnp
from jax import lax
from jax.experimental import pallas as pl
from jax.experimental.pallas import tpu as pltpu


def _chamfer_body(x1_ref, x2_ref, sq2_ref, idx1_ref, idx2_ref,
                  cmin_ref, cidx_ref, *, tn: int, m: int, nblk: int):
    k = pl.program_id(1)

    x1t = x1_ref[0]
    x2 = x2_ref[0]
    sq2r = sq2_ref[0]
    x2n = x2 * jnp.float32(-2.0)
    sq1t = jnp.sum(x1t * x1t, axis=1, keepdims=True)

    inner2 = lax.dot_general(x1t, x2n, (((1,), (1,)), ((), ())),
                             preferred_element_type=jnp.float32)
    d = (sq1t + sq2r) + inner2

    big = jnp.int32(m)

    lb = 128
    nlb = m // lb
    run_v = d[:, 0:lb]
    run_l = jnp.zeros((tn, lb), jnp.int32)
    for l in range(1, nlb):
        x = d[:, l * lb:(l + 1) * lb]
        win = x < run_v
        run_v = jnp.where(win, x, run_v)
        run_l = jnp.where(win, l, run_l)
    rmin = jnp.min(run_v, axis=1, keepdims=True)
    lane_iota = lax.broadcasted_iota(jnp.int32, (tn, lb), 1)
    ridx = jnp.min(jnp.where(run_v == rmin, run_l * lb + lane_iota, big),
                   axis=1, keepdims=True)
    idx1_ref[0] = ridx

    nsc = tn // 8
    d3 = d.reshape(nsc, 8, m)
    run_v2 = d3[0]
    run_c = jnp.zeros((8, m), jnp.int32)
    for c in range(1, nsc):
        x = d3[c]
        win = x < run_v2
        run_v2 = jnp.where(win, x, run_v2)
        run_c = jnp.where(win, c, run_c)
    cmin_t = jnp.min(run_v2, axis=0, keepdims=True)
    sub_iota = lax.broadcasted_iota(jnp.int32, (8, m), 0)
    cidx_t = jnp.min(jnp.where(run_v2 == cmin_t, run_c * 8 + sub_iota, big),
                     axis=0, keepdims=True) + k * tn

    @pl.when(k == 0)
    def _():
        cmin_ref[...] = cmin_t
        cidx_ref[...] = cidx_t

    @pl.when(k > 0)
    def _():
        prev_min = cmin_ref[...]
        prev_idx = cidx_ref[...]
        upd = cmin_t < prev_min
        cmin_ref[...] = jnp.where(upd, cmin_t, prev_min)
        cidx_ref[...] = jnp.where(upd, cidx_t, prev_idx)

    @pl.when(k == nblk - 1)
    def _():
        idx2_ref[0] = cidx_ref[...]


@jax.jit
def kernel(xyz1, xyz2):
    b, n, _ = xyz1.shape
    m = xyz2.shape[1]
    tn = 2048
    nblk = n // tn

    sq2 = jnp.sum(xyz2 * xyz2, axis=-1)[:, None, :]

    grid = (b, nblk)
    idx1, idx2 = pl.pallas_call(
        functools.partial(_chamfer_body, tn=tn, m=m, nblk=nblk),
        grid=grid,
        in_specs=[
            pl.BlockSpec((1, tn, 3), lambda bi, ki: (bi, ki, 0)),
            pl.BlockSpec((1, m, 3), lambda bi, ki: (bi, 0, 0)),
            pl.BlockSpec((1, 1, m), lambda bi, ki: (bi, 0, 0)),
        ],
        out_specs=[
            pl.BlockSpec((1, tn, 1), lambda bi, ki: (bi, ki, 0)),
            pl.BlockSpec((1, 1, m), lambda bi, ki: (bi, 0, 0)),
        ],
        out_shape=[
            jax.ShapeDtypeStruct((b, n, 1), jnp.int32),
            jax.ShapeDtypeStruct((b, 1, m), jnp.int32),
        ],
        scratch_shapes=[
            pltpu.VMEM((1, m), jnp.float32),
            pltpu.VMEM((1, m), jnp.int32),
        ],
        compiler_params=pltpu.CompilerParams(
            dimension_semantics=("arbitrary", "arbitrary"),
        ),
    )(xyz1, xyz2, sq2)

    return idx1[..., 0], idx2[:, 0, :]


# speedup vs baseline: 3.9175x; 1.1377x over previous
import functools

import jax
import jax.numpy as jnp
from jax import lax
from jax.experimental import pallas as pl
from jax.experimental.pallas import tpu as pltpu


def _chamfer_body(x1_ref, x2t_ref, idx1_ref, idx2_ref,
                  cmin_ref, cidx_ref, *, tn: int, m: int, nblk: int):
    k = pl.program_id(1)

    x1t = x1_ref[0]
    x2tn = x2t_ref[0]
    sq1t = jnp.sum(x1t * x1t, axis=1, keepdims=True)
    sq2r = jnp.sum(x2tn * x2tn, axis=0, keepdims=True) * jnp.float32(0.25)

    inner2 = lax.dot_general(x1t, x2tn, (((1,), (0,)), ((), ())),
                             preferred_element_type=jnp.float32)
    d = (sq1t + sq2r) + inner2

    big = jnp.int32(m)

    lb = 128
    nlb = m // lb
    run_v = d[:, 0:lb]
    run_l = jnp.zeros((tn, lb), jnp.int32)
    for l in range(1, nlb):
        x = d[:, l * lb:(l + 1) * lb]
        win = x < run_v
        run_v = jnp.where(win, x, run_v)
        run_l = jnp.where(win, l, run_l)
    v_t = jnp.swapaxes(run_v, 0, 1)
    l_t = jnp.swapaxes(run_l, 0, 1)
    rmin = jnp.min(v_t, axis=0, keepdims=True)
    sub128 = lax.broadcasted_iota(jnp.int32, (lb, tn), 0)
    ridx = jnp.min(jnp.where(v_t == rmin, l_t * lb + sub128, big),
                   axis=0, keepdims=True)
    idx1_ref[0] = ridx

    nsc = tn // 8
    d3 = d.reshape(nsc, 8, m)
    run_v2 = d3[0]
    run_c = jnp.zeros((8, m), jnp.int32)
    for c in range(1, nsc):
        x = d3[c]
        win = x < run_v2
        run_v2 = jnp.where(win, x, run_v2)
        run_c = jnp.where(win, c, run_c)
    cmin_t = jnp.min(run_v2, axis=0, keepdims=True)
    sub_iota = lax.broadcasted_iota(jnp.int32, (8, m), 0)
    cidx_t = jnp.min(jnp.where(run_v2 == cmin_t, run_c * 8 + sub_iota, big),
                     axis=0, keepdims=True) + k * tn

    @pl.when(k == 0)
    def _():
        cmin_ref[...] = cmin_t
        cidx_ref[...] = cidx_t

    @pl.when(k > 0)
    def _():
        prev_min = cmin_ref[...]
        prev_idx = cidx_ref[...]
        upd = cmin_t < prev_min
        cmin_ref[...] = jnp.where(upd, cmin_t, prev_min)
        cidx_ref[...] = jnp.where(upd, cidx_t, prev_idx)

    @pl.when(k == nblk - 1)
    def _():
        idx2_ref[0] = cidx_ref[...]


@jax.jit
def kernel(xyz1, xyz2):
    b, n, _ = xyz1.shape
    m = xyz2.shape[1]
    tn = 2048
    nblk = n // tn

    x2tn = jnp.swapaxes(xyz2 * jnp.float32(-2.0), 1, 2)

    grid = (b, nblk)
    idx1, idx2 = pl.pallas_call(
        functools.partial(_chamfer_body, tn=tn, m=m, nblk=nblk),
        grid=grid,
        in_specs=[
            pl.BlockSpec((1, tn, 3), lambda bi, ki: (bi, ki, 0)),
            pl.BlockSpec((1, 3, m), lambda bi, ki: (bi, 0, 0)),
        ],
        out_specs=[
            pl.BlockSpec((1, 1, tn), lambda bi, ki: (bi, 0, ki)),
            pl.BlockSpec((1, 1, m), lambda bi, ki: (bi, 0, 0)),
        ],
        out_shape=[
            jax.ShapeDtypeStruct((b, 1, n), jnp.int32),
            jax.ShapeDtypeStruct((b, 1, m), jnp.int32),
        ],
        scratch_shapes=[
            pltpu.VMEM((1, m), jnp.float32),
            pltpu.VMEM((1, m), jnp.int32),
        ],
        compiler_params=pltpu.CompilerParams(
            dimension_semantics=("arbitrary", "arbitrary"),
        ),
    )(xyz1, x2tn)

    return idx1[:, 0, :], idx2[:, 0, :]


# speedup vs baseline: 4.0395x; 1.0311x over previous
import functools

import jax
import jax.numpy as jnp
from jax import lax
from jax.experimental import pallas as pl
from jax.experimental.pallas import tpu as pltpu


def _chamfer_body(x1_ref, x2t_ref, idx1_ref, idx2_ref,
                  cmin_ref, cidx_ref, *, tn: int, m: int, nblk: int):
    k = pl.program_id(1)

    x1t = x1_ref[0]
    x2tn = x2t_ref[0]
    sq1t = jnp.sum(x1t * x1t, axis=1, keepdims=True)
    sq2r = jnp.sum(x2tn * x2tn, axis=0, keepdims=True) * jnp.float32(0.25)

    inner2 = lax.dot_general(x1t, x2tn, (((1,), (0,)), ((), ())),
                             preferred_element_type=jnp.float32)
    d = (sq1t + sq2r) + inner2

    big = jnp.int32(m)

    lb = 128
    nlb = m // lb
    run_v = d[:, 0:lb]
    run_l = jnp.zeros((tn, lb), jnp.int32)
    for l in range(1, nlb):
        x = d[:, l * lb:(l + 1) * lb]
        win = x < run_v
        run_v = jnp.minimum(x, run_v)
        run_l = jnp.where(win, l, run_l)
    v_t = jnp.swapaxes(run_v, 0, 1)
    l_t = jnp.swapaxes(run_l, 0, 1)
    rmin = jnp.min(v_t, axis=0, keepdims=True)
    sub128 = lax.broadcasted_iota(jnp.int32, (lb, tn), 0)
    ridx = jnp.min(jnp.where(v_t == rmin, l_t * lb + sub128, big),
                   axis=0, keepdims=True)
    idx1_ref[0] = ridx

    nsc = tn // 8
    d3 = d.reshape(nsc, 8, m)
    run_v2 = d3[0]
    run_c = jnp.zeros((8, m), jnp.int32)
    for c in range(1, nsc):
        x = d3[c]
        win = x < run_v2
        run_v2 = jnp.minimum(x, run_v2)
        run_c = jnp.where(win, c, run_c)
    cmin_t = jnp.min(run_v2, axis=0, keepdims=True)
    sub_iota = lax.broadcasted_iota(jnp.int32, (8, m), 0)
    cidx_t = jnp.min(jnp.where(run_v2 == cmin_t, run_c * 8 + sub_iota, big),
                     axis=0, keepdims=True) + k * tn

    @pl.when(k == 0)
    def _():
        cmin_ref[...] = cmin_t
        cidx_ref[...] = cidx_t

    @pl.when(k > 0)
    def _():
        prev_min = cmin_ref[...]
        prev_idx = cidx_ref[...]
        upd = cmin_t < prev_min
        cmin_ref[...] = jnp.where(upd, cmin_t, prev_min)
        cidx_ref[...] = jnp.where(upd, cidx_t, prev_idx)

    @pl.when(k == nblk - 1)
    def _():
        idx2_ref[0] = cidx_ref[...]


@jax.jit
def kernel(xyz1, xyz2):
    b, n, _ = xyz1.shape
    m = xyz2.shape[1]
    tn = 2048
    nblk = n // tn

    x2tn = jnp.swapaxes(xyz2 * jnp.float32(-2.0), 1, 2)

    grid = (b, nblk)
    idx1, idx2 = pl.pallas_call(
        functools.partial(_chamfer_body, tn=tn, m=m, nblk=nblk),
        grid=grid,
        in_specs=[
            pl.BlockSpec((1, tn, 3), lambda bi, ki: (bi, ki, 0)),
            pl.BlockSpec((1, 3, m), lambda bi, ki: (bi, 0, 0)),
        ],
        out_specs=[
            pl.BlockSpec((1, 1, tn), lambda bi, ki: (bi, 0, ki)),
            pl.BlockSpec((1, 1, m), lambda bi, ki: (bi, 0, 0)),
        ],
        out_shape=[
            jax.ShapeDtypeStruct((b, 1, n), jnp.int32),
            jax.ShapeDtypeStruct((b, 1, m), jnp.int32),
        ],
        scratch_shapes=[
            pltpu.VMEM((1, m), jnp.float32),
            pltpu.VMEM((1, m), jnp.int32),
        ],
        compiler_params=pltpu.CompilerParams(
            dimension_semantics=("arbitrary", "arbitrary"),
        ),
    )(xyz1, x2tn)

    return idx1[:, 0, :], idx2[:, 0, :]
